# baseline (speedup 1.0000x reference)
import jax
import jax.numpy as jnp
import numpy as np
from jax.experimental import pallas as pl
from jax.experimental.pallas import tpu as pltpu

_D = 20
_H = 16
_BT = 2048

_EYE = np.eye(_D, dtype=np.float32)
_N3 = np.stack([np.roll(_EYE, r - 1, axis=0) for r in range(3)])
_N2 = np.stack([np.roll(_EYE, r, axis=0) for r in range(2)])


def _band_proj(W3, b3, offsets):
    Pmask = []
    for k, off in enumerate(offsets):
        m = np.zeros((_D, _D * _D), np.float32)
        for i in range(_D):
            m[i, _D * i + (i + off) % _D] = 1.0
        Pmask.append(m)
    Pm = jnp.asarray(np.stack(Pmask))
    P = jnp.einsum("kic,uk->iuc", Pm, W3[:, 1:])
    bias = jnp.einsum("kic,k->c", Pm, b3[1:])[None]
    return P.reshape(_D * _H, _D * _D), bias


def _f_proj(W3, b3):
    P = jnp.kron(jnp.asarray(_EYE), W3[:, 0:1])
    bias = jnp.broadcast_to(b3[0], (1, _D))
    return P, bias


def _body(x_ref, a1a_ref, a1b_ref, k1a_ref, k1b_ref, k2a_ref, k2b_ref,
          pg1_ref, pg2_ref, pf1_ref, pf2_ref, b1a_ref, b1b_ref, b2a_ref,
          b2b_ref, b3a_ref, b3b_ref, bg1_ref, bg2_ref, bf1_ref, bf2_ref,
          f1_ref, g1_ref, f2_ref, g2_ref):
    f32 = jnp.float32
    bf16 = jnp.bfloat16
    xb = x_ref[...]

    h = jnp.maximum(jnp.dot(xb, a1a_ref[...], preferred_element_type=f32) + b1a_ref[...], 0.0)
    h = jnp.maximum(jnp.dot(h.astype(bf16), k1a_ref[...].astype(bf16), preferred_element_type=f32) + b2a_ref[...], 0.0)
    h = jnp.maximum(jnp.dot(h.astype(bf16), k2a_ref[...].astype(bf16), preferred_element_type=f32) + b3a_ref[...], 0.0)
    hb = h.astype(bf16)
    g1_ref[...] = jnp.dot(hb, pg1_ref[...].astype(bf16), preferred_element_type=f32) + bg1_ref[...]
    f1_ref[...] = jnp.dot(hb, pf1_ref[...].astype(bf16), preferred_element_type=f32) + bf1_ref[...]

    h = jnp.maximum(jnp.dot(xb, a1b_ref[...], preferred_element_type=f32) + b1b_ref[...], 0.0)
    h = jnp.maximum(jnp.dot(h.astype(bf16), k1b_ref[...].astype(bf16), preferred_element_type=f32) + b2b_ref[...], 0.0)
    h = jnp.maximum(jnp.dot(h.astype(bf16), k2b_ref[...].astype(bf16), preferred_element_type=f32) + b3b_ref[...], 0.0)
    hb = h.astype(bf16)
    g2_ref[...] = jnp.dot(hb, pg2_ref[...].astype(bf16), preferred_element_type=f32) + bg2_ref[...]
    f2_ref[...] = jnp.dot(hb, pf2_ref[...].astype(bf16), preferred_element_type=f32) + bf2_ref[...]


def kernel(x, Wa0, ba0, Wa1, ba1, Wa2, ba2, Wa3, ba3,
           Wb0, bb0, Wb1, bb1, Wb2, bb2, Wb3, bb3):
    f32 = jnp.float32
    B = x.shape[0]
    eye = jnp.asarray(_EYE)

    a1a = jnp.einsum("rki,rc->kic", jnp.asarray(_N3), Wa0).reshape(_D, _D * _H)
    a1b = jnp.einsum("rki,rc->kic", jnp.asarray(_N2), Wb0).reshape(_D, _D * _H)
    k1a = jnp.kron(eye, Wa1)
    k1b = jnp.kron(eye, Wb1)
    k2a = jnp.kron(eye, Wa2)
    k2b = jnp.kron(eye, Wb2)
    pg1, bg1 = _band_proj(Wa3, ba3, (-1, 0))
    pg2, bg2 = _band_proj(Wb3, bb3, (-1, 0, 1))
    pf1, bf1 = _f_proj(Wa3, ba3)
    pf2, bf2 = _f_proj(Wb3, bb3)
    b1a = jnp.tile(ba0, _D)[None]
    b1b = jnp.tile(bb0, _D)[None]
    b2a = jnp.tile(ba1, _D)[None]
    b2b = jnp.tile(bb1, _D)[None]
    b3a = jnp.tile(ba2, _D)[None]
    b3b = jnp.tile(bb2, _D)[None]

    consts = (a1a, a1b, k1a, k1b, k2a, k2b, pg1, pg2, pf1, pf2,
              b1a, b1b, b2a, b2b, b3a, b3b, bg1, bg2, bf1, bf2)

    grid = (B // _BT,)
    in_specs = [pl.BlockSpec((_BT, _D), lambda b: (b, 0))]
    in_specs += [pl.BlockSpec(c.shape, lambda b: (0, 0)) for c in consts]
    out_specs = [
        pl.BlockSpec((_BT, _D), lambda b: (b, 0)),
        pl.BlockSpec((_BT, _D * _D), lambda b: (b, 0)),
        pl.BlockSpec((_BT, _D), lambda b: (b, 0)),
        pl.BlockSpec((_BT, _D * _D), lambda b: (b, 0)),
    ]
    out_shape = [
        jax.ShapeDtypeStruct((B, _D), f32),
        jax.ShapeDtypeStruct((B, _D * _D), f32),
        jax.ShapeDtypeStruct((B, _D), f32),
        jax.ShapeDtypeStruct((B, _D * _D), f32),
    ]
    f1, g1, f2, g2 = pl.pallas_call(
        _body, grid=grid, in_specs=in_specs, out_specs=out_specs,
        out_shape=out_shape,
        compiler_params=pltpu.CompilerParams(
            dimension_semantics=("arbitrary",)))(x, *consts)
    return (f1[:, :, None], g1.reshape(B, _D, _D),
            f2[:, :, None], g2.reshape(B, _D, _D))
